# baseline (speedup 1.0000x reference)
import functools

import jax
import jax.numpy as jnp
from jax import lax
from jax.experimental import pallas as pl
from jax.experimental.pallas import tpu as pltpu
from jax.experimental.pallas import tpu_sc as plsc

N_S, N_C = 50000, 10000
D = 128

NCORES, NSUB = 2, 16
NW = NCORES * NSUB

E_UND, E_TEA = 400000, 200000
G = 8
UB = 104
TB = 56
E_UND_P = NW * UB * 128
E_TEA_P = NW * TB * 128

NR = 5
RNG = 10000
R_TAB = 10240
R_STRIPE = R_TAB // NSUB
DUMP = 10200
ZROWS = 32

R_BLK = 400


def _mesh():
    return plsc.VectorSubcoreMesh(core_axis_name="c", subcore_axis_name="s")


def _sc_agg(src_c, src_s, und_src, und_dst, tea_src, tea_dst, tok,
            with_deg):
    out_type = [
        jax.ShapeDtypeStruct((NCORES, NR, R_TAB, D), jnp.float32),
        jax.ShapeDtypeStruct((NCORES, R_TAB, D), jnp.float32),
        jax.ShapeDtypeStruct((128,), jnp.float32),
    ]
    if with_deg:
        out_type += [
            jax.ShapeDtypeStruct((NCORES, NR, R_TAB, D), jnp.float32),
            jax.ShapeDtypeStruct((NCORES, R_TAB, D), jnp.float32),
        ]

    @functools.partial(
        pl.kernel,
        out_type=tuple(out_type),
        mesh=_mesh(),
        scratch_types=[
            pltpu.VMEM((G, 128), jnp.int32),
            pltpu.VMEM((G, 128), jnp.int32),
            pltpu.VMEM((128, D), jnp.float32),
            pltpu.VMEM((ZROWS, D), jnp.float32),
            pltpu.VMEM((128,), jnp.float32),
            pltpu.VMEM_SHARED((R_TAB, D), jnp.float32),
        ],
    )
    def k(srcc_hbm, srcs_hbm, us_hbm, ud_hbm, ts_hbm, td_hbm, tok_hbm,
          aggs_out, aggc_out, tok_out, *rest):
        if with_deg:
            degs_out, degc_out = rest[0], rest[1]
            rest = rest[2:]
        sidx_v, didx_v, rows0, zeros_v, tok_v, tab_sh = rest
        cid = lax.axis_index("c")
        sid = lax.axis_index("s")
        wid = sid * NCORES + cid

        pltpu.sync_copy(tok_hbm, tok_v)
        zero16 = jnp.zeros((16,), jnp.float32)

        def zrow(i, _):
            for t in range(D // 16):
                zeros_v[i, pl.ds(t * 16, 16)] = zero16
            return 0
        lax.fori_loop(0, ZROWS, zrow, 0)

        if with_deg:
            one16 = jnp.ones((16,), jnp.float32)

            def orow(i, _):
                for t in range(D // 16):
                    rows0[i, pl.ds(t * 16, 16)] = one16
                return 0
            lax.fori_loop(0, 128, orow, 0)

        def load_and_xform(src_hbm, dst_hbm, g, base, do_src):
            if do_src:
                pltpu.sync_copy(src_hbm.at[wid, pl.ds(g * G, G)], sidx_v)
            pltpu.sync_copy(dst_hbm.at[wid, pl.ds(g * G, G)], didx_v)

            def xform(i, _):
                j, t = i // 8, i % 8
                v = didx_v[j, pl.ds(t * 16, 16)] - base
                ok = (v >= 0) & (v < RNG)
                didx_v[j, pl.ds(t * 16, 16)] = jnp.where(ok, v, DUMP)
                return 0
            lax.fori_loop(0, G * 8, xform, 0)

        def zero_table():
            for q in range(R_STRIPE // ZROWS):
                pltpu.sync_copy(
                    zeros_v,
                    tab_sh.at[pl.ds(sid * R_STRIPE + q * ZROWS, ZROWS)])
            plsc.subcore_barrier()

        def gather_pass(src_hbm, dst_hbm, table, ngroups, base):
            zero_table()

            def grp(g, _):
                load_and_xform(src_hbm, dst_hbm, g, base, True)
                for j in range(G):
                    pltpu.sync_copy(table.at[sidx_v.at[j]], rows0)
                    pltpu.sync_copy(rows0, tab_sh.at[didx_v.at[j]], add=True)
                return 0
            lax.fori_loop(0, ngroups, grp, 0)
            plsc.subcore_barrier()

        def ones_pass(dst_hbm, ngroups, base):
            zero_table()

            def grp(g, _):
                load_and_xform(None, dst_hbm, g, base, False)
                for j in range(G):
                    pltpu.sync_copy(rows0, tab_sh.at[didx_v.at[j]], add=True)
                return 0
            lax.fori_loop(0, ngroups, grp, 0)
            plsc.subcore_barrier()

        def copy_out(out4, r):
            if r is None:
                dst = out4.at[cid, pl.ds(sid * R_STRIPE, R_STRIPE)]
            else:
                dst = out4.at[cid, r, pl.ds(sid * R_STRIPE, R_STRIPE)]
            pltpu.sync_copy(tab_sh.at[pl.ds(sid * R_STRIPE, R_STRIPE)], dst)

        if with_deg:
            for r in range(NR):
                ones_pass(ud_hbm, UB // G, r * RNG)
                copy_out(degs_out, r)
            ones_pass(td_hbm, TB // G, 0)
            copy_out(degc_out, None)

        for r in range(NR):
            gather_pass(us_hbm, ud_hbm, srcc_hbm, UB // G, r * RNG)
            copy_out(aggs_out, r)

        gather_pass(ts_hbm, td_hbm, srcs_hbm, TB // G, 0)
        copy_out(aggc_out, None)

        @pl.when((sid == 0) & (cid == 0))
        def _():
            pltpu.sync_copy(tok_v, tok_out)

    return k(src_c, src_s, und_src, und_dst, tea_src, tea_dst, tok)


def _proj0_body(x_ref, wfc_ref, bfc_ref, wself_ref, bself_ref, wn_ref,
                self_ref, src_ref):
    h = jnp.dot(x_ref[...], wfc_ref[...],
                preferred_element_type=jnp.float32) + bfc_ref[...]
    self_ref[...] = jnp.dot(h, wself_ref[...],
                            preferred_element_type=jnp.float32) + bself_ref[...]
    src_ref[...] = jnp.dot(h, wn_ref[...], preferred_element_type=jnp.float32)


def _proj0(x, wfc, bfc, wself, bself, wneigh, n):
    nb = n // R_BLK
    return pl.pallas_call(
        _proj0_body,
        grid=(nb,),
        in_specs=[
            pl.BlockSpec((R_BLK, D), lambda b: (b, 0)),
            pl.BlockSpec((D, D), lambda b: (0, 0)),
            pl.BlockSpec((1, D), lambda b: (0, 0)),
            pl.BlockSpec((D, D), lambda b: (0, 0)),
            pl.BlockSpec((1, D), lambda b: (0, 0)),
            pl.BlockSpec((D, D), lambda b: (0, 0)),
        ],
        out_specs=[
            pl.BlockSpec((R_BLK, D), lambda b: (b, 0)),
            pl.BlockSpec((R_BLK, D), lambda b: (b, 0)),
        ],
        out_shape=[
            jax.ShapeDtypeStruct((n, D), jnp.float32),
            jax.ShapeDtypeStruct((n, D), jnp.float32),
        ],
    )(x, wfc, bfc.reshape(1, D), wself, bself.reshape(1, D), wneigh)


def _invdeg_body(parts_ref, out_ref):
    d = jnp.sum(parts_ref[...], axis=tuple(range(parts_ref.ndim - 2)))[:, 0:1]
    out_ref[...] = 1.0 / jnp.maximum(d, 1.0)


def _invdeg(parts, student):
    blk = 1000
    if student:
        grid = (NR * (RNG // blk),)
        in_spec = pl.BlockSpec(
            (NCORES, 1, blk, D), lambda b: (0, b // 10, b % 10, 0))
        n = NR * RNG
    else:
        grid = (RNG // blk,)
        in_spec = pl.BlockSpec((NCORES, blk, D), lambda b: (0, b, 0))
        n = RNG
    return pl.pallas_call(
        _invdeg_body,
        grid=grid,
        in_specs=[in_spec],
        out_specs=pl.BlockSpec((blk, 1), lambda b: (b, 0)),
        out_shape=jax.ShapeDtypeStruct((n, 1), jnp.float32),
    )(parts)


def _agg_spec_s():
    return pl.BlockSpec((NCORES, 1, R_BLK, D), lambda b: (0, b // 25, b % 25, 0))


def _agg_spec_c():
    return pl.BlockSpec((NCORES, R_BLK, D), lambda b: (0, b, 0))


def _combineproj_body(self_ref, agg_ref, deg_ref, wself_ref, bself_ref,
                      wn_ref, selfo_ref, src_ref):
    a = jnp.sum(agg_ref[...], axis=tuple(range(agg_ref.ndim - 2)))
    h = jnp.maximum(self_ref[...] + a * deg_ref[...], 0.0)
    selfo_ref[...] = jnp.dot(h, wself_ref[...],
                             preferred_element_type=jnp.float32) + bself_ref[...]
    src_ref[...] = jnp.dot(h, wn_ref[...], preferred_element_type=jnp.float32)


def _combineproj(selfv, agg, deg, wself, bself, wneigh, n, student):
    nb = n // R_BLK
    return pl.pallas_call(
        _combineproj_body,
        grid=(nb,),
        in_specs=[
            pl.BlockSpec((R_BLK, D), lambda b: (b, 0)),
            _agg_spec_s() if student else _agg_spec_c(),
            pl.BlockSpec((R_BLK, 1), lambda b: (b, 0)),
            pl.BlockSpec((D, D), lambda b: (0, 0)),
            pl.BlockSpec((1, D), lambda b: (0, 0)),
            pl.BlockSpec((D, D), lambda b: (0, 0)),
        ],
        out_specs=[
            pl.BlockSpec((R_BLK, D), lambda b: (b, 0)),
            pl.BlockSpec((R_BLK, D), lambda b: (b, 0)),
        ],
        out_shape=[
            jax.ShapeDtypeStruct((n, D), jnp.float32),
            jax.ShapeDtypeStruct((n, D), jnp.float32),
        ],
    )(selfv, agg, deg, wself, bself.reshape(1, D), wneigh)


def _final_body(self_ref, agg_ref, deg_ref, out_ref):
    a = jnp.sum(agg_ref[...], axis=tuple(range(agg_ref.ndim - 2)))
    out_ref[...] = self_ref[...] + a * deg_ref[...]


def _final(selfv, agg, deg, n, student):
    nb = n // R_BLK
    return pl.pallas_call(
        _final_body,
        grid=(nb,),
        in_specs=[
            pl.BlockSpec((R_BLK, D), lambda b: (b, 0)),
            _agg_spec_s() if student else _agg_spec_c(),
            pl.BlockSpec((R_BLK, 1), lambda b: (b, 0)),
        ],
        out_specs=pl.BlockSpec((R_BLK, D), lambda b: (b, 0)),
        out_shape=jax.ShapeDtypeStruct((n, D), jnp.float32),
    )(selfv, agg, deg)


def kernel(x_student, x_concept, x_lecture, src_understands, dst_understands,
           src_teaches, dst_teaches, params):
    del x_lecture

    i32 = jnp.int32
    us = jnp.concatenate(
        [src_understands.astype(i32),
         jnp.zeros((E_UND_P - E_UND,), i32)]).reshape(NW, UB, 128)
    ud = jnp.concatenate(
        [dst_understands.astype(i32),
         jnp.full((E_UND_P - E_UND,), N_S, i32)]).reshape(NW, UB, 128)
    ts = jnp.concatenate(
        [src_teaches.astype(i32),
         jnp.zeros((E_TEA_P - E_TEA,), i32)]).reshape(NW, TB, 128)
    td = jnp.concatenate(
        [dst_teaches.astype(i32),
         jnp.full((E_TEA_P - E_TEA,), N_C, i32)]).reshape(NW, TB, 128)

    L = params["layers"]
    self_s, src_s = _proj0(
        x_student, params["fc_student"]["W"], params["fc_student"]["b"],
        L[0]["und"]["W_self"], L[0]["und"]["b"], L[0]["tea"]["W_neigh"], N_S)
    self_c, src_c = _proj0(
        x_concept, params["fc_concept"]["W"], params["fc_concept"]["b"],
        L[0]["tea"]["W_self"], L[0]["tea"]["b"], L[0]["und"]["W_neigh"], N_C)

    tok = jnp.zeros((128,), jnp.float32)
    agg_s, agg_c, tok, degp_s, degp_c = _sc_agg(
        src_c, src_s, us, ud, ts, td, tok, True)
    deg_s = _invdeg(degp_s, True)[:N_S]
    deg_c = _invdeg(degp_c, False)[:N_C]

    for i in range(2):
        if i > 0:
            agg_s, agg_c, tok = _sc_agg(src_c, src_s, us, ud, ts, td, tok,
                                        False)
        nxt = L[i + 1]
        self_s, src_s = _combineproj(
            self_s, agg_s, deg_s,
            nxt["und"]["W_self"], nxt["und"]["b"], nxt["tea"]["W_neigh"],
            N_S, True)
        self_c, src_c = _combineproj(
            self_c, agg_c, deg_c,
            nxt["tea"]["W_self"], nxt["tea"]["b"], nxt["und"]["W_neigh"],
            N_C, False)

    agg_s, agg_c, tok = _sc_agg(src_c, src_s, us, ud, ts, td, tok, False)
    out_s = _final(self_s, agg_s, deg_s, N_S, True)
    out_c = _final(self_c, agg_c, deg_c, N_C, False)
    return out_s, out_c


# speedup vs baseline: 1.0013x; 1.0013x over previous
import functools

import jax
import jax.numpy as jnp
from jax import lax
from jax.experimental import pallas as pl
from jax.experimental.pallas import tpu as pltpu
from jax.experimental.pallas import tpu_sc as plsc

N_S, N_C = 50000, 10000
D = 128

NCORES, NSUB = 2, 16
NW = NCORES * NSUB

E_UND, E_TEA = 400000, 200000
G = 8
UB = 104
TB = 56
E_UND_P = NW * UB * 128
E_TEA_P = NW * TB * 128

NR = 5
RNG = 10000
R_TAB = 10112
R_STRIPE = R_TAB // NSUB
DUMP = 10050

R_BLK = 400


def _mesh():
    return plsc.VectorSubcoreMesh(core_axis_name="c", subcore_axis_name="s")


def _sc_agg(src_c, src_s, und_src, und_dst, tea_src, tea_dst, tok,
            with_deg):
    out_type = [
        jax.ShapeDtypeStruct((NCORES, NR, R_TAB, D), jnp.float32),
        jax.ShapeDtypeStruct((NCORES, R_TAB, D), jnp.float32),
        jax.ShapeDtypeStruct((128,), jnp.float32),
    ]
    if with_deg:
        out_type += [
            jax.ShapeDtypeStruct((NCORES, NR, R_TAB, D), jnp.float32),
            jax.ShapeDtypeStruct((NCORES, R_TAB, D), jnp.float32),
        ]

    @functools.partial(
        pl.kernel,
        out_type=tuple(out_type),
        mesh=_mesh(),
        scratch_types=[
            pltpu.VMEM((G, 128), jnp.int32),
            pltpu.VMEM((G, 128), jnp.int32),
            pltpu.VMEM((128, D), jnp.float32),
            pltpu.VMEM((128, D), jnp.float32),
            pltpu.VMEM((128,), jnp.float32),
            pltpu.VMEM_SHARED((R_TAB, D), jnp.float32),
            pltpu.SemaphoreType.DMA,
            pltpu.SemaphoreType.DMA,
        ],
    )
    def k(srcc_hbm, srcs_hbm, us_hbm, ud_hbm, ts_hbm, td_hbm, tok_hbm,
          aggs_out, aggc_out, tok_out, *rest):
        if with_deg:
            degs_out, degc_out = rest[0], rest[1]
            rest = rest[2:]
        sidx_v, didx_v, rows0, rows1, tok_v, tab_sh, sem0, sem1 = rest
        cid = lax.axis_index("c")
        sid = lax.axis_index("s")
        wid = sid * NCORES + cid

        pltpu.sync_copy(tok_hbm, tok_v)

        def fill(buf, val16):
            def row(i, _):
                for t in range(D // 16):
                    buf[i, pl.ds(t * 16, 16)] = val16
                return 0
            lax.fori_loop(0, 128, row, 0)

        def load_and_xform(src_hbm, dst_hbm, g, base, do_src):
            if do_src:
                pltpu.sync_copy(src_hbm.at[wid, pl.ds(g * G, G)], sidx_v)
            pltpu.sync_copy(dst_hbm.at[wid, pl.ds(g * G, G)], didx_v)

            def xform(i, _):
                j, t = i // 8, i % 8
                v = didx_v[j, pl.ds(t * 16, 16)] - base
                ok = (v >= 0) & (v < RNG)
                didx_v[j, pl.ds(t * 16, 16)] = jnp.where(ok, v, DUMP)
                return 0
            lax.fori_loop(0, G * 8, xform, 0)

        def zero_table():
            fill(rows0, jnp.zeros((16,), jnp.float32))
            for q in range(R_STRIPE // 128):
                pltpu.sync_copy(
                    rows0, tab_sh.at[pl.ds(sid * R_STRIPE + q * 128, 128)])
            rem = R_STRIPE % 128
            if rem:
                pltpu.sync_copy(
                    rows0.at[pl.ds(0, rem)],
                    tab_sh.at[pl.ds(
                        sid * R_STRIPE + (R_STRIPE // 128) * 128, rem)])
            plsc.subcore_barrier()

        def gather_pass(src_hbm, dst_hbm, table, ngroups, base):
            zero_table()
            bufs = (rows0, rows1)
            sems = (sem0, sem1)

            def grp(g, _):
                load_and_xform(src_hbm, dst_hbm, g, base, True)
                pltpu.async_copy(table.at[sidx_v.at[0]], rows0, sem0)
                pltpu.async_copy(table.at[sidx_v.at[1]], rows1, sem1)
                for j in range(G):
                    b, s = bufs[j % 2], sems[j % 2]
                    pltpu.make_async_copy(
                        table.at[sidx_v.at[j]], b, s).wait()
                    pltpu.sync_copy(b, tab_sh.at[didx_v.at[j]], add=True)
                    if j + 2 < G:
                        pltpu.async_copy(
                            table.at[sidx_v.at[j + 2]], b, s)
                return 0
            lax.fori_loop(0, ngroups, grp, 0)
            plsc.subcore_barrier()

        def ones_pass(dst_hbm, ngroups, base):
            zero_table()
            fill(rows1, jnp.ones((16,), jnp.float32))

            def grp(g, _):
                load_and_xform(None, dst_hbm, g, base, False)
                for j in range(G):
                    pltpu.sync_copy(rows1, tab_sh.at[didx_v.at[j]], add=True)
                return 0
            lax.fori_loop(0, ngroups, grp, 0)
            plsc.subcore_barrier()

        def copy_out(out4, r):
            if r is None:
                dst = out4.at[cid, pl.ds(sid * R_STRIPE, R_STRIPE)]
            else:
                dst = out4.at[cid, r, pl.ds(sid * R_STRIPE, R_STRIPE)]
            pltpu.sync_copy(tab_sh.at[pl.ds(sid * R_STRIPE, R_STRIPE)], dst)

        if with_deg:
            for r in range(NR):
                ones_pass(ud_hbm, UB // G, r * RNG)
                copy_out(degs_out, r)
            ones_pass(td_hbm, TB // G, 0)
            copy_out(degc_out, None)

        for r in range(NR):
            gather_pass(us_hbm, ud_hbm, srcc_hbm, UB // G, r * RNG)
            copy_out(aggs_out, r)

        gather_pass(ts_hbm, td_hbm, srcs_hbm, TB // G, 0)
        copy_out(aggc_out, None)

        @pl.when((sid == 0) & (cid == 0))
        def _():
            pltpu.sync_copy(tok_v, tok_out)

    return k(src_c, src_s, und_src, und_dst, tea_src, tea_dst, tok)


def _proj0_body(x_ref, wfc_ref, bfc_ref, wself_ref, bself_ref, wn_ref,
                self_ref, src_ref):
    h = jnp.dot(x_ref[...], wfc_ref[...],
                preferred_element_type=jnp.float32) + bfc_ref[...]
    self_ref[...] = jnp.dot(h, wself_ref[...],
                            preferred_element_type=jnp.float32) + bself_ref[...]
    src_ref[...] = jnp.dot(h, wn_ref[...], preferred_element_type=jnp.float32)


def _proj0(x, wfc, bfc, wself, bself, wneigh, n):
    nb = n // R_BLK
    return pl.pallas_call(
        _proj0_body,
        grid=(nb,),
        in_specs=[
            pl.BlockSpec((R_BLK, D), lambda b: (b, 0)),
            pl.BlockSpec((D, D), lambda b: (0, 0)),
            pl.BlockSpec((1, D), lambda b: (0, 0)),
            pl.BlockSpec((D, D), lambda b: (0, 0)),
            pl.BlockSpec((1, D), lambda b: (0, 0)),
            pl.BlockSpec((D, D), lambda b: (0, 0)),
        ],
        out_specs=[
            pl.BlockSpec((R_BLK, D), lambda b: (b, 0)),
            pl.BlockSpec((R_BLK, D), lambda b: (b, 0)),
        ],
        out_shape=[
            jax.ShapeDtypeStruct((n, D), jnp.float32),
            jax.ShapeDtypeStruct((n, D), jnp.float32),
        ],
    )(x, wfc, bfc.reshape(1, D), wself, bself.reshape(1, D), wneigh)


def _invdeg_body(parts_ref, out_ref):
    d = jnp.sum(parts_ref[...], axis=tuple(range(parts_ref.ndim - 2)))[:, 0:1]
    out_ref[...] = 1.0 / jnp.maximum(d, 1.0)


def _invdeg(parts, student):
    blk = 1000
    if student:
        grid = (NR * (RNG // blk),)
        in_spec = pl.BlockSpec(
            (NCORES, 1, blk, D), lambda b: (0, b // 10, b % 10, 0))
        n = NR * RNG
    else:
        grid = (RNG // blk,)
        in_spec = pl.BlockSpec((NCORES, blk, D), lambda b: (0, b, 0))
        n = RNG
    return pl.pallas_call(
        _invdeg_body,
        grid=grid,
        in_specs=[in_spec],
        out_specs=pl.BlockSpec((blk, 1), lambda b: (b, 0)),
        out_shape=jax.ShapeDtypeStruct((n, 1), jnp.float32),
    )(parts)


def _agg_spec_s():
    return pl.BlockSpec((NCORES, 1, R_BLK, D), lambda b: (0, b // 25, b % 25, 0))


def _agg_spec_c():
    return pl.BlockSpec((NCORES, R_BLK, D), lambda b: (0, b, 0))


def _combineproj_body(self_ref, agg_ref, deg_ref, wself_ref, bself_ref,
                      wn_ref, selfo_ref, src_ref):
    a = jnp.sum(agg_ref[...], axis=tuple(range(agg_ref.ndim - 2)))
    h = jnp.maximum(self_ref[...] + a * deg_ref[...], 0.0)
    selfo_ref[...] = jnp.dot(h, wself_ref[...],
                             preferred_element_type=jnp.float32) + bself_ref[...]
    src_ref[...] = jnp.dot(h, wn_ref[...], preferred_element_type=jnp.float32)


def _combineproj(selfv, agg, deg, wself, bself, wneigh, n, student):
    nb = n // R_BLK
    return pl.pallas_call(
        _combineproj_body,
        grid=(nb,),
        in_specs=[
            pl.BlockSpec((R_BLK, D), lambda b: (b, 0)),
            _agg_spec_s() if student else _agg_spec_c(),
            pl.BlockSpec((R_BLK, 1), lambda b: (b, 0)),
            pl.BlockSpec((D, D), lambda b: (0, 0)),
            pl.BlockSpec((1, D), lambda b: (0, 0)),
            pl.BlockSpec((D, D), lambda b: (0, 0)),
        ],
        out_specs=[
            pl.BlockSpec((R_BLK, D), lambda b: (b, 0)),
            pl.BlockSpec((R_BLK, D), lambda b: (b, 0)),
        ],
        out_shape=[
            jax.ShapeDtypeStruct((n, D), jnp.float32),
            jax.ShapeDtypeStruct((n, D), jnp.float32),
        ],
    )(selfv, agg, deg, wself, bself.reshape(1, D), wneigh)


def _final_body(self_ref, agg_ref, deg_ref, out_ref):
    a = jnp.sum(agg_ref[...], axis=tuple(range(agg_ref.ndim - 2)))
    out_ref[...] = self_ref[...] + a * deg_ref[...]


def _final(selfv, agg, deg, n, student):
    nb = n // R_BLK
    return pl.pallas_call(
        _final_body,
        grid=(nb,),
        in_specs=[
            pl.BlockSpec((R_BLK, D), lambda b: (b, 0)),
            _agg_spec_s() if student else _agg_spec_c(),
            pl.BlockSpec((R_BLK, 1), lambda b: (b, 0)),
        ],
        out_specs=pl.BlockSpec((R_BLK, D), lambda b: (b, 0)),
        out_shape=jax.ShapeDtypeStruct((n, D), jnp.float32),
    )(selfv, agg, deg)


def kernel(x_student, x_concept, x_lecture, src_understands, dst_understands,
           src_teaches, dst_teaches, params):
    del x_lecture

    i32 = jnp.int32
    us = jnp.concatenate(
        [src_understands.astype(i32),
         jnp.zeros((E_UND_P - E_UND,), i32)]).reshape(NW, UB, 128)
    ud = jnp.concatenate(
        [dst_understands.astype(i32),
         jnp.full((E_UND_P - E_UND,), N_S, i32)]).reshape(NW, UB, 128)
    ts = jnp.concatenate(
        [src_teaches.astype(i32),
         jnp.zeros((E_TEA_P - E_TEA,), i32)]).reshape(NW, TB, 128)
    td = jnp.concatenate(
        [dst_teaches.astype(i32),
         jnp.full((E_TEA_P - E_TEA,), N_C, i32)]).reshape(NW, TB, 128)

    L = params["layers"]
    self_s, src_s = _proj0(
        x_student, params["fc_student"]["W"], params["fc_student"]["b"],
        L[0]["und"]["W_self"], L[0]["und"]["b"], L[0]["tea"]["W_neigh"], N_S)
    self_c, src_c = _proj0(
        x_concept, params["fc_concept"]["W"], params["fc_concept"]["b"],
        L[0]["tea"]["W_self"], L[0]["tea"]["b"], L[0]["und"]["W_neigh"], N_C)

    tok = jnp.zeros((128,), jnp.float32)
    agg_s, agg_c, tok, degp_s, degp_c = _sc_agg(
        src_c, src_s, us, ud, ts, td, tok, True)
    deg_s = _invdeg(degp_s, True)[:N_S]
    deg_c = _invdeg(degp_c, False)[:N_C]

    for i in range(2):
        if i > 0:
            agg_s, agg_c, tok = _sc_agg(src_c, src_s, us, ud, ts, td, tok,
                                        False)
        nxt = L[i + 1]
        self_s, src_s = _combineproj(
            self_s, agg_s, deg_s,
            nxt["und"]["W_self"], nxt["und"]["b"], nxt["tea"]["W_neigh"],
            N_S, True)
        self_c, src_c = _combineproj(
            self_c, agg_c, deg_c,
            nxt["tea"]["W_self"], nxt["tea"]["b"], nxt["und"]["W_neigh"],
            N_C, False)

    agg_s, agg_c, tok = _sc_agg(src_c, src_s, us, ud, ts, td, tok, False)
    out_s = _final(self_s, agg_s, deg_s, N_S, True)
    out_c = _final(self_c, agg_c, deg_c, N_C, False)
    return out_s, out_c
